# baseline (speedup 1.0000x reference)
import functools

import jax
import jax.numpy as jnp
from jax import lax
from jax.experimental import pallas as pl
from jax.experimental.pallas import tpu as pltpu
from jax.experimental.pallas import tpu_sc as plsc

_HI = jax.lax.Precision.HIGHEST


_N_PAD = 50176
_WROWS = 12544
_ACC_R = 12672
_TRASH_D = 50176
_CH = 4096
_NCH = 25
_R_TILE = _CH * _NCH
_E_PAD = 16 * _R_TILE


def _sc_agg_body(dst_hbm, src_hbm, xt_hbm, zrows_hbm, out_hbm,
                 pairs_d, pairs_s, dstidx, srcidx, rows, acc, sem):
    core = lax.axis_index("c")
    tile = lax.axis_index("s")
    base_t = tile * _R_TILE
    soff = core * 50000
    zpt = _ACC_R // 16

    def window_body(wi, carry):
        lo = wi * _WROWS
        pltpu.sync_copy(zrows_hbm, acc.at[pl.ds(tile * zpt, zpt)])
        plsc.subcore_barrier()

        def chunk_body(c, carry2):
            cb = base_t + c * _CH
            pltpu.sync_copy(dst_hbm.at[pl.ds(cb, _CH)], pairs_d)
            pltpu.sync_copy(src_hbm.at[pl.ds(cb, _CH)], pairs_s)

            def batch_body(b, carry3):
                for k in range(8):
                    d = pairs_d[pl.ds(b * 128 + k * 16, 16)]
                    dl = d - lo
                    inw = (dl >= 0) & (dl < _WROWS)
                    dstidx[pl.ds(k * 16, 16)] = jnp.where(inw, dl, _WROWS)
                    srcidx[pl.ds(k * 16, 16)] = (
                        pairs_s[pl.ds(b * 128 + k * 16, 16)] + soff)
                pltpu.async_copy(xt_hbm.at[srcidx], rows, sem).wait()
                pltpu.sync_copy(rows, acc.at[dstidx], add=True)
                return carry3

            lax.fori_loop(0, _CH // 128, batch_body, jnp.int32(0))
            return carry2

        lax.fori_loop(0, _NCH, chunk_body, jnp.int32(0))
        plsc.subcore_barrier()
        dpt = _WROWS // 16
        pltpu.sync_copy(acc.at[pl.ds(tile * dpt, dpt)],
                        out_hbm.at[core].at[pl.ds(lo + tile * dpt, dpt)])
        plsc.subcore_barrier()
        return carry

    lax.fori_loop(0, 4, window_body, jnp.int32(0))


def _sc_aggregate(dst_list, src_list, x1r):
    xt = jnp.concatenate([x1r[:, :128], x1r[:, 128:]], axis=0)
    zrows = jnp.zeros((_ACC_R // 16, 128), jnp.float32)
    mesh = plsc.VectorSubcoreMesh(core_axis_name="c", subcore_axis_name="s")
    f = functools.partial(
        pl.kernel, mesh=mesh,
        out_type=jax.ShapeDtypeStruct((2, _N_PAD, 128), jnp.float32),
        scratch_types=[
            pltpu.VMEM((_CH,), jnp.int32),
            pltpu.VMEM((_CH,), jnp.int32),
            pltpu.VMEM((128,), jnp.int32),
            pltpu.VMEM((128,), jnp.int32),
            pltpu.VMEM((128, 128), jnp.float32),
            pltpu.VMEM_SHARED((_ACC_R, 128), jnp.float32),
            pltpu.SemaphoreType.DMA,
        ],
    )(_sc_agg_body)
    out = f(dst_list, src_list, xt, zrows)
    return jnp.concatenate([out[0], out[1]], axis=1)


def _rnd_bf16(a):
    return jax.lax.reduce_precision(a, exponent_bits=8, mantissa_bits=7)


def _mm_hi(a, b):
    return jnp.dot(a, b, precision=_HI, preferred_element_type=jnp.float32)


def _mm_bf(a, b):
    return jnp.dot(a.astype(jnp.bfloat16), b.astype(jnp.bfloat16),
                   preferred_element_type=jnp.float32)


def _mlp_body(m_ref, w2_ref, b2_ref, w3_ref, b3_ref, out_ref, h_ref):
    @pl.when(pl.program_id(0) == 0)
    def _():
        h = _mm_bf(m_ref[...], w2_ref[...].T)
        h_ref[...] = jax.nn.relu(h + b2_ref[...])
    y = _mm_bf(h_ref[...], w3_ref[...].T)
    out_ref[...] = jax.nn.sigmoid(y + b3_ref[...])


def _decoder(m, lin2_w, lin2_b, lin3_w, lin3_b):
    OUT = lin3_w.shape[0]
    TILE = 2048
    grid = OUT // TILE
    return pl.pallas_call(
        _mlp_body,
        grid=(grid,),
        in_specs=[
            pl.BlockSpec((1, 512), lambda i: (0, 0)),
            pl.BlockSpec((1024, 512), lambda i: (0, 0)),
            pl.BlockSpec((1, 1024), lambda i: (0, 0)),
            pl.BlockSpec((TILE, 1024), lambda i: (i, 0)),
            pl.BlockSpec((1, TILE), lambda i: (0, i)),
        ],
        out_specs=pl.BlockSpec((1, TILE), lambda i: (0, i)),
        out_shape=jax.ShapeDtypeStruct((1, OUT), jnp.float32),
        scratch_shapes=[pltpu.VMEM((1, 1024), jnp.float32)],
    )(m.reshape(1, 512), lin2_w, lin2_b.reshape(1, 1024), lin3_w,
      lin3_b.reshape(1, OUT))


def kernel(verts, edges, gc1_w0, gc1_b0, gc1_w1, gc1_b1,
           gc2_w0, gc2_b0, gc2_w1, gc2_b1,
           lin2_w, lin2_b, lin3_w, lin3_b):
    n = verts.shape[0]
    e0, e1 = edges[:, 0], edges[:, 1]

    vx = jnp.concatenate([verts, jnp.ones((n, 1), jnp.float32)], axis=1)
    agg4 = jnp.zeros((n, 4), jnp.float32)
    agg4 = agg4.at[e0].add(vx[e1]).at[e1].add(vx[e0])
    aggv, deg = agg4[:, :3], agg4[:, 3:4]

    x1 = jax.nn.relu(_mm_hi(verts, gc1_w0.T) + gc1_b0 + _mm_hi(aggv, gc1_w1.T)
                     + deg * gc1_b1)
    x1r = _rnd_bf16(x1)

    npad = _E_PAD - 2 * e0.shape[0]
    dst_list = jnp.concatenate([e0, e1, jnp.full((npad,), _TRASH_D, jnp.int32)])
    src_list = jnp.concatenate([e1, e0, jnp.zeros((npad,), jnp.int32)])
    aggx = _sc_aggregate(dst_list, src_list, x1r)[:n]
    x2 = jax.nn.relu(_mm_bf(x1r, gc2_w0.T) + gc2_b0
                     + _mm_hi(aggx, _rnd_bf16(gc2_w1).T) + deg * gc2_b1)

    m = jnp.max(x2, axis=0)
    img = _decoder(m, lin2_w, lin2_b, lin3_w, lin3_b)
    img = img.reshape(1, 3, 128, 128)
    img = jnp.repeat(jnp.repeat(img, 4, axis=2), 4, axis=3)
    return img


# speedup vs baseline: 1.5789x; 1.5789x over previous
import functools

import jax
import jax.numpy as jnp
from jax import lax
from jax.experimental import pallas as pl
from jax.experimental.pallas import tpu as pltpu
from jax.experimental.pallas import tpu_sc as plsc

_HI = jax.lax.Precision.HIGHEST


_N_PAD = 50176
_ACC_R = 50304
_TRASH_D = 50176
_CH = 4096
_NCH = 25
_R_TILE = _CH * _NCH
_E_PAD = 16 * _R_TILE


def _sc_agg_body(gbase, dst_hbm, src_hbm, xt_hbm, zrows_hbm, out_hbm,
                 pairs_d, pairs_s, dstidx, srcidx, rows, acc, sem):
    core = lax.axis_index("c")
    tile = lax.axis_index("s")
    base_t = tile * _R_TILE
    rows_pt = _ACC_R // 16
    soff = (2 * gbase + core) * 50000

    pltpu.sync_copy(zrows_hbm, acc.at[pl.ds(tile * rows_pt, rows_pt)])
    plsc.subcore_barrier()

    def chunk_body(c, carry2):
        cb = base_t + c * _CH
        pltpu.sync_copy(dst_hbm.at[pl.ds(cb, _CH)], pairs_d)
        pltpu.sync_copy(src_hbm.at[pl.ds(cb, _CH)], pairs_s)

        def batch_body(b, carry3):
            for k in range(8):
                dstidx[pl.ds(k * 16, 16)] = pairs_d[pl.ds(b * 128 + k * 16, 16)]
                srcidx[pl.ds(k * 16, 16)] = (
                    pairs_s[pl.ds(b * 128 + k * 16, 16)] + soff)
            pltpu.async_copy(xt_hbm.at[srcidx], rows, sem).wait()
            pltpu.sync_copy(rows, acc.at[dstidx], add=True)
            return carry3

        lax.fori_loop(0, _CH // 128, batch_body, jnp.int32(0))
        return carry2

    lax.fori_loop(0, _NCH, chunk_body, jnp.int32(0))
    plsc.subcore_barrier()
    dr = pl.ds(tile * (_N_PAD // 16), _N_PAD // 16)
    pltpu.sync_copy(acc.at[dr], out_hbm.at[core].at[dr])


def _sc_aggregate(dst_list, src_list, x1r):
    xt = x1r.reshape(50000, 8, 32).transpose(1, 0, 2).reshape(8 * 50000, 32)
    zrows = jnp.zeros((_ACC_R // 16, 32), jnp.float32)
    mesh = plsc.VectorSubcoreMesh(core_axis_name="c", subcore_axis_name="s")
    parts = []
    for g in range(4):
        f = functools.partial(
            pl.kernel, mesh=mesh,
            out_type=jax.ShapeDtypeStruct((2, _ACC_R, 32), jnp.float32),
            scratch_types=[
                pltpu.VMEM((_CH,), jnp.int32),
                pltpu.VMEM((_CH,), jnp.int32),
                pltpu.VMEM((128,), jnp.int32),
                pltpu.VMEM((128,), jnp.int32),
                pltpu.VMEM((128, 32), jnp.float32),
                pltpu.VMEM_SHARED((_ACC_R, 32), jnp.float32),
                pltpu.SemaphoreType.DMA,
            ],
        )(functools.partial(_sc_agg_body, g))
        parts.append(f(dst_list, src_list, xt, zrows))
    full = jnp.stack(parts)[:, :, :_N_PAD, :]
    return full.transpose(2, 0, 1, 3).reshape(_N_PAD, 256)


def _rnd_bf16(a):
    return jax.lax.reduce_precision(a, exponent_bits=8, mantissa_bits=7)


def _mm_hi(a, b):
    return jnp.dot(a, b, precision=_HI, preferred_element_type=jnp.float32)


def _mm_bf(a, b):
    return jnp.dot(a.astype(jnp.bfloat16), b.astype(jnp.bfloat16),
                   preferred_element_type=jnp.float32)


def _mlp_body(m_ref, w2_ref, b2_ref, w3_ref, b3_ref, out_ref, h_ref):
    @pl.when(pl.program_id(0) == 0)
    def _():
        h = _mm_bf(m_ref[...], w2_ref[...].T)
        h_ref[...] = jax.nn.relu(h + b2_ref[...])
    y = _mm_bf(h_ref[...], w3_ref[...].T)
    out_ref[...] = jax.nn.sigmoid(y + b3_ref[...])


def _decoder(m, lin2_w, lin2_b, lin3_w, lin3_b):
    OUT = lin3_w.shape[0]
    TILE = 2048
    grid = OUT // TILE
    return pl.pallas_call(
        _mlp_body,
        grid=(grid,),
        in_specs=[
            pl.BlockSpec((1, 512), lambda i: (0, 0)),
            pl.BlockSpec((1024, 512), lambda i: (0, 0)),
            pl.BlockSpec((1, 1024), lambda i: (0, 0)),
            pl.BlockSpec((TILE, 1024), lambda i: (i, 0)),
            pl.BlockSpec((1, TILE), lambda i: (0, i)),
        ],
        out_specs=pl.BlockSpec((1, TILE), lambda i: (0, i)),
        out_shape=jax.ShapeDtypeStruct((1, OUT), jnp.float32),
        scratch_shapes=[pltpu.VMEM((1, 1024), jnp.float32)],
    )(m.reshape(1, 512), lin2_w, lin2_b.reshape(1, 1024), lin3_w,
      lin3_b.reshape(1, OUT))


def kernel(verts, edges, gc1_w0, gc1_b0, gc1_w1, gc1_b1,
           gc2_w0, gc2_b0, gc2_w1, gc2_b1,
           lin2_w, lin2_b, lin3_w, lin3_b):
    n = verts.shape[0]
    e0, e1 = edges[:, 0], edges[:, 1]

    vx = jnp.concatenate([verts, jnp.ones((n, 1), jnp.float32)], axis=1)
    agg4 = jnp.zeros((n, 4), jnp.float32)
    agg4 = agg4.at[e0].add(vx[e1]).at[e1].add(vx[e0])
    aggv, deg = agg4[:, :3], agg4[:, 3:4]

    x1 = jax.nn.relu(_mm_hi(verts, gc1_w0.T) + gc1_b0 + _mm_hi(aggv, gc1_w1.T)
                     + deg * gc1_b1)
    x1r = _rnd_bf16(x1)

    aggx = jnp.zeros_like(x1r)
    aggx = aggx.at[e0].add(x1r[e1]).at[e1].add(x1r[e0])
    x2 = jax.nn.relu(_mm_bf(x1r, gc2_w0.T) + gc2_b0
                     + _mm_hi(aggx, _rnd_bf16(gc2_w1).T) + deg * gc2_b1)

    m = jnp.max(x2, axis=0)
    img = _decoder(m, lin2_w, lin2_b, lin3_w, lin3_b)
    img = img.reshape(1, 3, 128, 128)
    img = jnp.repeat(jnp.repeat(img, 4, axis=2), 4, axis=3)
    return img
